# baseline (speedup 1.0000x reference)
import functools

import jax
import jax.numpy as jnp
from jax import lax
from jax.experimental import pallas as pl
from jax.experimental.pallas import tpu as pltpu
from jax.experimental.pallas import tpu_sc as plsc

_D = 768
_SEQ = 77
_BATCH = 1024
_NC = 2
_NS = 16
_NW = _NC * _NS
_ROWS = _BATCH * _SEQ
_RPW = _ROWS // _NW
_W = 32
_NWIN = _RPW // _W
_LANES = 16


def _sc_embed(ids_flat, tok_w, pos_w):
  mesh = plsc.VectorSubcoreMesh(core_axis_name="c", subcore_axis_name="s")

  @functools.partial(
      pl.kernel,
      mesh=mesh,
      out_type=jax.ShapeDtypeStruct((_ROWS, _D), jnp.float32),
      scratch_types=[
          pltpu.VMEM((_RPW,), jnp.int32),
          pltpu.VMEM((_SEQ, _D), jnp.float32),
          pltpu.VMEM((_W, _D), jnp.float32),
          pltpu.SemaphoreType.DMA,
      ],
  )
  def k(ids_hbm, tab_hbm, pos_hbm, out_hbm, idx_v, pos_v, buf_v, sem):
    wid = lax.axis_index("s") * _NC + lax.axis_index("c")
    base = wid * _RPW
    pltpu.sync_copy(ids_hbm.at[pl.ds(base, _RPW)], idx_v)
    pltpu.sync_copy(pos_hbm, pos_v)

    def win_body(kw, p0):
      pltpu.async_copy(
          tab_hbm.at[idx_v.at[pl.ds(kw * _W, _W)]], buf_v, sem
      ).wait()

      def row_body(i, p):
        for c in range(_D // _LANES):
          sl = pl.ds(c * _LANES, _LANES)
          buf_v[i, sl] = buf_v[i, sl] + pos_v[p, sl]
        p = p + 1
        return lax.select(p == _SEQ, 0, p)

      p_end = lax.fori_loop(0, _W, row_body, p0)
      pltpu.sync_copy(buf_v, out_hbm.at[pl.ds(base + kw * _W, _W)])
      return p_end

    lax.fori_loop(0, _NWIN, win_body, 0)

  return k(ids_flat, tok_w, pos_w)


def kernel(input_ids, token_embedding_weight, position_embedding_weight):
  ids = input_ids.astype(jnp.int32).reshape(-1)
  out = _sc_embed(ids, token_embedding_weight, position_embedding_weight)
  return out.reshape(_BATCH, _SEQ, _D)


# speedup vs baseline: 1.1362x; 1.1362x over previous
import functools

import jax
import jax.numpy as jnp
from jax import lax
from jax.experimental import pallas as pl
from jax.experimental.pallas import tpu as pltpu
from jax.experimental.pallas import tpu_sc as plsc

_D = 768
_SEQ = 77
_BATCH = 1024
_NC = 2
_NS = 16
_NW = _NC * _NS
_ROWS = _BATCH * _SEQ
_RPW = _ROWS // _NW
_W = 32
_NWIN = _RPW // _W
_LANES = 16


def _sc_embed(ids_flat, tok_w, pos_w):
  mesh = plsc.VectorSubcoreMesh(core_axis_name="c", subcore_axis_name="s")

  @functools.partial(
      pl.kernel,
      mesh=mesh,
      out_type=jax.ShapeDtypeStruct((_ROWS, _D), jnp.float32),
      scratch_types=[
          pltpu.VMEM((_RPW,), jnp.int32),
          pltpu.VMEM((_SEQ, _D), jnp.float32),
          pltpu.VMEM((_W, _D), jnp.float32),
          pltpu.VMEM((_W, _D), jnp.float32),
          pltpu.SemaphoreType.DMA,
          pltpu.SemaphoreType.DMA,
          pltpu.SemaphoreType.DMA,
          pltpu.SemaphoreType.DMA,
      ],
  )
  def k(ids_hbm, tab_hbm, pos_hbm, out_hbm, idx_v, pos_v, buf_a, buf_b,
        gsem_a, gsem_b, wsem_a, wsem_b):
    wid = lax.axis_index("s") * _NC + lax.axis_index("c")
    base = wid * _RPW
    pltpu.sync_copy(ids_hbm.at[pl.ds(base, _RPW)], idx_v)
    pltpu.sync_copy(pos_hbm, pos_v)

    def g_start(kw, buf, gsem):
      pltpu.async_copy(tab_hbm.at[idx_v.at[pl.ds(kw * _W, _W)]], buf, gsem)

    def g_wait(kw, buf, gsem):
      pltpu.make_async_copy(
          tab_hbm.at[idx_v.at[pl.ds(kw * _W, _W)]], buf, gsem
      ).wait()

    def w_start(kw, buf, wsem):
      pltpu.async_copy(buf, out_hbm.at[pl.ds(base + kw * _W, _W)], wsem)

    def w_wait(kw, buf, wsem):
      pltpu.make_async_copy(
          buf, out_hbm.at[pl.ds(base + kw * _W, _W)], wsem
      ).wait()

    def add_pos(buf, p0):
      def row_body(i, p):
        for c in range(_D // _LANES):
          sl = pl.ds(c * _LANES, _LANES)
          buf[i, sl] = buf[i, sl] + pos_v[p, sl]
        p = p + 1
        return lax.select(p == _SEQ, 0, p)

      return lax.fori_loop(0, _W, row_body, p0)

    g_start(0, buf_a, gsem_a)
    g_start(1, buf_b, gsem_b)

    npair = _NWIN // 2

    def pair_body(t, p0):
      k0 = 2 * t
      g_wait(k0, buf_a, gsem_a)
      p1 = add_pos(buf_a, p0)
      w_start(k0, buf_a, wsem_a)
      g_wait(k0 + 1, buf_b, gsem_b)
      p2 = add_pos(buf_b, p1)
      w_start(k0 + 1, buf_b, wsem_b)

      @pl.when(k0 + 2 < _NWIN)
      def _():
        w_wait(k0, buf_a, wsem_a)
        g_start(k0 + 2, buf_a, gsem_a)

      @pl.when(k0 + 3 < _NWIN)
      def _():
        w_wait(k0 + 1, buf_b, wsem_b)
        g_start(k0 + 3, buf_b, gsem_b)

      return p2

    p = lax.fori_loop(0, npair, pair_body, 0)

    kw_last = _NWIN - 1
    g_wait(kw_last, buf_a, gsem_a)
    add_pos(buf_a, p)
    w_start(kw_last, buf_a, wsem_a)
    w_wait(kw_last, buf_a, wsem_a)
    w_wait(kw_last - 1, buf_b, wsem_b)

  return k(ids_flat, tok_w, pos_w)


def kernel(input_ids, token_embedding_weight, position_embedding_weight):
  ids = input_ids.astype(jnp.int32).reshape(-1)
  out = _sc_embed(ids, token_embedding_weight, position_embedding_weight)
  return out.reshape(_BATCH, _SEQ, _D)


# speedup vs baseline: 1.8783x; 1.6531x over previous
import functools

import jax
import jax.numpy as jnp
from jax import lax
from jax.experimental import pallas as pl
from jax.experimental.pallas import tpu as pltpu
from jax.experimental.pallas import tpu_sc as plsc

_D = 768
_SEQ = 77
_BATCH = 1024
_NC = 2
_NS = 16
_NW = _NC * _NS
_ROWS = _BATCH * _SEQ
_RPW = _ROWS // _NW
_W = 32
_NWIN = _RPW // _W
_LANES = 16


def _sc_embed(ids_flat, tok_w, pos_w):
  mesh = plsc.VectorSubcoreMesh(core_axis_name="c", subcore_axis_name="s")

  @functools.partial(
      pl.kernel,
      mesh=mesh,
      out_type=jax.ShapeDtypeStruct((_ROWS, _D), jnp.float32),
      scratch_types=[
          pltpu.VMEM((_RPW,), jnp.int32),
          pltpu.VMEM((_SEQ * _D,), jnp.float32),
          pltpu.VMEM((_W, _D), jnp.float32),
          pltpu.VMEM((_W, _D), jnp.float32),
          pltpu.SemaphoreType.DMA,
          pltpu.SemaphoreType.DMA,
          pltpu.SemaphoreType.DMA,
          pltpu.SemaphoreType.DMA,
      ],
  )
  def k(ids_hbm, tab_hbm, pos_hbm, out_hbm, idx_v, pos_v, buf_a, buf_b,
        gsem_a, gsem_b, wsem_a, wsem_b):
    wid = lax.axis_index("s") * _NC + lax.axis_index("c")
    base = wid * _RPW
    pltpu.sync_copy(ids_hbm.at[pl.ds(base, _RPW)], idx_v)
    pltpu.sync_copy(pos_hbm, pos_v)

    def g_start(kw, buf, gsem):
      pltpu.async_copy(tab_hbm.at[idx_v.at[pl.ds(kw * _W, _W)]], buf, gsem)

    def g_wait(kw, buf, gsem):
      pltpu.make_async_copy(
          tab_hbm.at[idx_v.at[pl.ds(kw * _W, _W)]], buf, gsem
      ).wait()

    def w_start(kw, buf, wsem):
      pltpu.async_copy(buf, out_hbm.at[pl.ds(base + kw * _W, _W)], wsem)

    def w_wait(kw, buf, wsem):
      pltpu.make_async_copy(
          buf, out_hbm.at[pl.ds(base + kw * _W, _W)], wsem
      ).wait()

    def add_pos(buf, p0):
      @plsc.parallel_loop(0, _W)
      def _(i):
        p = p0 + i
        p = lax.select(p >= _SEQ, p - _SEQ, p)
        pbase = p * _D
        for c in range(_D // _LANES):
          sl = pl.ds(c * _LANES, _LANES)
          buf[i, sl] = buf[i, sl] + pos_v[pl.ds(pbase + c * _LANES, _LANES)]

      p_end = p0 + _W
      return lax.select(p_end >= _SEQ, p_end - _SEQ, p_end)

    g_start(0, buf_a, gsem_a)
    g_start(1, buf_b, gsem_b)

    npair = _NWIN // 2

    def pair_body(t, p0):
      k0 = 2 * t
      g_wait(k0, buf_a, gsem_a)
      p1 = add_pos(buf_a, p0)
      w_start(k0, buf_a, wsem_a)
      g_wait(k0 + 1, buf_b, gsem_b)
      p2 = add_pos(buf_b, p1)
      w_start(k0 + 1, buf_b, wsem_b)

      @pl.when(k0 + 2 < _NWIN)
      def _():
        w_wait(k0, buf_a, wsem_a)
        g_start(k0 + 2, buf_a, gsem_a)

      @pl.when(k0 + 3 < _NWIN)
      def _():
        w_wait(k0 + 1, buf_b, wsem_b)
        g_start(k0 + 3, buf_b, gsem_b)

      return p2

    p = lax.fori_loop(0, npair, pair_body, 0)

    kw_last = _NWIN - 1
    g_wait(kw_last, buf_a, gsem_a)
    add_pos(buf_a, p)
    w_start(kw_last, buf_a, wsem_a)
    w_wait(kw_last, buf_a, wsem_a)
    w_wait(kw_last - 1, buf_b, wsem_b)

  return k(ids_flat, tok_w, pos_w)


def kernel(input_ids, token_embedding_weight, position_embedding_weight):
  ids = input_ids.astype(jnp.int32).reshape(-1)
  pos_flat = position_embedding_weight.reshape(-1)
  out = _sc_embed(ids, token_embedding_weight, pos_flat)
  return out.reshape(_BATCH, _SEQ, _D)


# speedup vs baseline: 2.4769x; 1.3187x over previous
import functools

import jax
import jax.numpy as jnp
from jax import lax
from jax.experimental import pallas as pl
from jax.experimental.pallas import tpu as pltpu
from jax.experimental.pallas import tpu_sc as plsc

_D = 768
_SEQ = 77
_BATCH = 1024
_NC = 2
_NS = 16
_NW = _NC * _NS
_SPW = _BATCH // _NW
_LANES = 16
_MAIN = 64
_TAIL0 = _MAIN - 3
_G = _MAIN + _LANES


def _sc_embed(ids_g, tok_w, pos_sw):
  mesh = plsc.VectorSubcoreMesh(core_axis_name="c", subcore_axis_name="s")

  @functools.partial(
      pl.kernel,
      mesh=mesh,
      out_type=jax.ShapeDtypeStruct((_BATCH, _SEQ, _D), jnp.float32),
      scratch_types=[
          pltpu.VMEM((_SPW * _G,), jnp.int32),
          pltpu.VMEM((_SEQ * _D // 2,), jnp.int32),
          pltpu.VMEM((_SEQ, _D), jnp.float32),
          pltpu.VMEM((_LANES, _D), jnp.float32),
          pltpu.SemaphoreType.DMA,
          pltpu.SemaphoreType.DMA,
          pltpu.SemaphoreType.DMA,
      ],
  )
  def k(ids_hbm, tab_hbm, pos_hbm, out_hbm, idx_v, pos_v, obuf, gbuf,
        gsem_m, gsem_t, wsem):
    wid = lax.axis_index("s") * _NC + lax.axis_index("c")
    seq0 = wid * _SPW
    pltpu.sync_copy(ids_hbm.at[pl.ds(seq0 * _G, _SPW * _G)], idx_v)
    pltpu.sync_copy(pos_hbm, pos_v)

    def gm_start(j):
      pltpu.async_copy(
          tab_hbm.at[idx_v.at[pl.ds(j * _G, _MAIN)]],
          obuf.at[pl.ds(0, _MAIN)], gsem_m)

    def gm_wait(j):
      pltpu.make_async_copy(
          tab_hbm.at[idx_v.at[pl.ds(j * _G, _MAIN)]],
          obuf.at[pl.ds(0, _MAIN)], gsem_m).wait()

    def gt_start(j):
      pltpu.async_copy(
          tab_hbm.at[idx_v.at[pl.ds(j * _G + _MAIN, _LANES)]], gbuf, gsem_t)

    def gt_wait(j):
      pltpu.make_async_copy(
          tab_hbm.at[idx_v.at[pl.ds(j * _G + _MAIN, _LANES)]], gbuf,
          gsem_t).wait()

    def w_start(j):
      pltpu.async_copy(obuf, out_hbm.at[seq0 + j], wsem)

    def w_wait(j):
      pltpu.make_async_copy(obuf, out_hbm.at[seq0 + j], wsem).wait()

    def pos_chunks(i, cp):
      packed = pos_v[pl.ds(i * (_D // 2) + _LANES * cp, _LANES)]
      lo = lax.bitcast_convert_type(packed << 16, jnp.float32)
      hi = lax.bitcast_convert_type(packed & jnp.int32(-65536), jnp.float32)
      return lo, hi

    def add_main():
      @plsc.parallel_loop(0, _MAIN)
      def _(i):
        for cp in range(_D // (2 * _LANES)):
          lo, hi = pos_chunks(i, cp)
          sl0 = pl.ds(2 * _LANES * cp, _LANES)
          sl1 = pl.ds(2 * _LANES * cp + _LANES, _LANES)
          obuf[i, sl0] = obuf[i, sl0] + lo
          obuf[i, sl1] = obuf[i, sl1] + hi

    def add_tail():
      @plsc.parallel_loop(_MAIN, _SEQ)
      def _(i):
        g = i - _TAIL0
        for cp in range(_D // (2 * _LANES)):
          lo, hi = pos_chunks(i, cp)
          sl0 = pl.ds(2 * _LANES * cp, _LANES)
          sl1 = pl.ds(2 * _LANES * cp + _LANES, _LANES)
          obuf[i, sl0] = gbuf[g, sl0] + lo
          obuf[i, sl1] = gbuf[g, sl1] + hi

    gm_start(0)
    gt_start(0)

    def seq_body(j, carry):
      gt_wait(j)
      add_tail()
      gt_start_next = j + 1 < _SPW

      @pl.when(gt_start_next)
      def _():
        gt_start(j + 1)

      gm_wait(j)
      add_main()
      w_start(j)

      @pl.when(gt_start_next)
      def _():
        w_wait(j)
        gm_start(j + 1)

      return carry

    lax.fori_loop(0, _SPW, seq_body, 0)
    w_wait(_SPW - 1)

  return k(ids_g, tok_w, pos_sw)


def kernel(input_ids, token_embedding_weight, position_embedding_weight):
  ids = input_ids.astype(jnp.int32)
  ids_g = jnp.concatenate([ids[:, :_MAIN], ids[:, _TAIL0:]], axis=1)
  ids_g = ids_g.reshape(-1)
  bits = lax.bitcast_convert_type(
      position_embedding_weight.astype(jnp.bfloat16), jnp.uint16
  ).reshape(-1, 2, _LANES).astype(jnp.uint32)
  pos_sw = lax.bitcast_convert_type(
      bits[:, 0, :] | (bits[:, 1, :] << 16), jnp.int32).reshape(-1)
  return _sc_embed(ids_g, token_embedding_weight, pos_sw)


# speedup vs baseline: 5.4104x; 2.1843x over previous
import functools

import jax
import jax.numpy as jnp
from jax import lax
from jax.experimental import pallas as pl
from jax.experimental.pallas import tpu as pltpu
from jax.experimental.pallas import tpu_sc as plsc

_D = 768
_SEQ = 77
_BATCH = 1024
_NC = 2
_NS = 16
_NW = _NC * _NS
_BPW = _BATCH // _NW
_LANES = 16
_PPW = _D // (2 * _LANES)


def _sc_embed(ids_w, tok_w, pos_pk):
  mesh = plsc.VectorSubcoreMesh(core_axis_name="c", subcore_axis_name="s")

  @functools.partial(
      pl.kernel,
      mesh=mesh,
      out_type=jax.ShapeDtypeStruct((_SEQ, _BATCH, _D), jnp.float32),
      scratch_types=[
          pltpu.VMEM((_SEQ * _BPW,), jnp.int32),
          pltpu.VMEM((_SEQ * _PPW * _LANES,), jnp.int32),
          pltpu.VMEM((_BPW, _D), jnp.float32),
          pltpu.VMEM((_BPW, _D), jnp.float32),
          pltpu.SemaphoreType.DMA,
          pltpu.SemaphoreType.DMA,
          pltpu.SemaphoreType.DMA,
          pltpu.SemaphoreType.DMA,
      ],
  )
  def k(ids_hbm, tab_hbm, pos_hbm, out_hbm, idx_v, pos_v, buf_a, buf_b,
        gsem_a, gsem_b, wsem_a, wsem_b):
    wid = lax.axis_index("s") * _NC + lax.axis_index("c")
    b0 = wid * _BPW
    pltpu.sync_copy(ids_hbm.at[pl.ds(wid * _SEQ * _BPW, _SEQ * _BPW)], idx_v)
    pltpu.sync_copy(pos_hbm, pos_v)

    def g_start(s, buf, gsem):
      pltpu.async_copy(tab_hbm.at[idx_v.at[pl.ds(s * _BPW, _BPW)]], buf, gsem)

    def g_wait(s, buf, gsem):
      pltpu.make_async_copy(
          tab_hbm.at[idx_v.at[pl.ds(s * _BPW, _BPW)]], buf, gsem).wait()

    def w_start(s, buf, wsem):
      pltpu.async_copy(buf, out_hbm.at[s, pl.ds(b0, _BPW)], wsem)

    def w_wait(s, buf, wsem):
      pltpu.make_async_copy(buf, out_hbm.at[s, pl.ds(b0, _BPW)], wsem).wait()

    def add_pos(s, buf):
      pchunks = []
      for cp in range(_PPW):
        packed = pos_v[pl.ds(s * (_D // 2) + _LANES * cp, _LANES)]
        pchunks.append(lax.bitcast_convert_type(packed << 16, jnp.float32))
        pchunks.append(
            lax.bitcast_convert_type(packed & jnp.int32(-65536), jnp.float32))

      @plsc.parallel_loop(0, _BPW)
      def _(i):
        for c in range(_D // _LANES):
          sl = pl.ds(c * _LANES, _LANES)
          buf[i, sl] = buf[i, sl] + pchunks[c]

    g_start(0, buf_a, gsem_a)
    g_start(1, buf_b, gsem_b)

    npair = _SEQ // 2

    def pair_body(t, carry):
      s = 2 * t
      g_wait(s, buf_a, gsem_a)
      add_pos(s, buf_a)
      w_start(s, buf_a, wsem_a)
      g_wait(s + 1, buf_b, gsem_b)
      add_pos(s + 1, buf_b)
      w_start(s + 1, buf_b, wsem_b)

      @pl.when(s + 2 < _SEQ)
      def _():
        w_wait(s, buf_a, wsem_a)
        g_start(s + 2, buf_a, gsem_a)

      @pl.when(s + 3 < _SEQ)
      def _():
        w_wait(s + 1, buf_b, wsem_b)
        g_start(s + 3, buf_b, gsem_b)

      return carry

    lax.fori_loop(0, npair, pair_body, 0)

    s_last = _SEQ - 1
    g_wait(s_last, buf_a, gsem_a)
    add_pos(s_last, buf_a)
    w_start(s_last, buf_a, wsem_a)
    w_wait(s_last, buf_a, wsem_a)
    w_wait(s_last - 1, buf_b, wsem_b)

  return k(ids_w, tok_w, pos_pk)


def kernel(input_ids, token_embedding_weight, position_embedding_weight):
  ids = input_ids.astype(jnp.int32)
  ids_w = ids.T.reshape(_SEQ, _NW, _BPW).transpose(1, 0, 2).reshape(-1)
  bits = lax.bitcast_convert_type(
      position_embedding_weight.astype(jnp.bfloat16), jnp.uint16
  ).reshape(-1, 2, _LANES).astype(jnp.uint32)
  pos_pk = lax.bitcast_convert_type(
      bits[:, 0, :] | (bits[:, 1, :] << 16), jnp.int32).reshape(-1)
  out_t = _sc_embed(ids_w, token_embedding_weight, pos_pk)
  return out_t.transpose(1, 0, 2)


# speedup vs baseline: 5.6296x; 1.0405x over previous
import functools

import jax
import jax.numpy as jnp
from jax import lax
from jax.experimental import pallas as pl
from jax.experimental.pallas import tpu as pltpu
from jax.experimental.pallas import tpu_sc as plsc

_D = 768
_SEQ = 77
_BATCH = 1024
_NC = 2
_NS = 16
_NW = _NC * _NS
_BPW = _BATCH // _NW
_LANES = 16
_PPW = _D // (2 * _LANES)


def _sc_embed(ids_w, tok_w, pos_pk):
  mesh = plsc.VectorSubcoreMesh(core_axis_name="c", subcore_axis_name="s")

  @functools.partial(
      pl.kernel,
      mesh=mesh,
      out_type=jax.ShapeDtypeStruct((_SEQ, _BATCH, _D), jnp.float32),
      scratch_types=[
          pltpu.VMEM((_SEQ * _BPW,), jnp.int32),
          pltpu.VMEM((_SEQ * _D,), jnp.float32),
          pltpu.VMEM((_BPW, _D), jnp.float32),
          pltpu.VMEM((_BPW, _D), jnp.float32),
          pltpu.SemaphoreType.DMA,
          pltpu.SemaphoreType.DMA,
          pltpu.SemaphoreType.DMA,
          pltpu.SemaphoreType.DMA,
      ],
  )
  def k(ids_hbm, tab_hbm, pos_hbm, out_hbm, idx_v, pos_v, buf_a, buf_b,
        gsem_a, gsem_b, wsem_a, wsem_b):
    wid = lax.axis_index("s") * _NC + lax.axis_index("c")
    b0 = wid * _BPW
    pltpu.sync_copy(ids_hbm.at[pl.ds(wid * _SEQ * _BPW, _SEQ * _BPW)], idx_v)
    pltpu.sync_copy(pos_hbm, pos_v)

    def g_start(s, buf, gsem):
      pltpu.async_copy(tab_hbm.at[idx_v.at[pl.ds(s * _BPW, _BPW)]], buf, gsem)

    def g_wait(s, buf, gsem):
      pltpu.make_async_copy(
          tab_hbm.at[idx_v.at[pl.ds(s * _BPW, _BPW)]], buf, gsem).wait()

    def w_start(s, buf, wsem):
      pltpu.async_copy(buf, out_hbm.at[s, pl.ds(b0, _BPW)], wsem)

    def w_wait(s, buf, wsem):
      pltpu.make_async_copy(buf, out_hbm.at[s, pl.ds(b0, _BPW)], wsem).wait()

    def add_pos(s, buf):
      pchunks = [
          pos_v[pl.ds(s * _D + c * _LANES, _LANES)]
          for c in range(_D // _LANES)
      ]

      @plsc.parallel_loop(0, _BPW)
      def _(i):
        for c in range(_D // _LANES):
          sl = pl.ds(c * _LANES, _LANES)
          buf[i, sl] = buf[i, sl] + pchunks[c]

    g_start(0, buf_a, gsem_a)
    g_start(1, buf_b, gsem_b)

    npair = _SEQ // 2

    def pair_body(t, carry):
      s = 2 * t
      g_wait(s, buf_a, gsem_a)
      add_pos(s, buf_a)
      w_start(s, buf_a, wsem_a)
      g_wait(s + 1, buf_b, gsem_b)
      add_pos(s + 1, buf_b)
      w_start(s + 1, buf_b, wsem_b)

      @pl.when(s + 2 < _SEQ)
      def _():
        w_wait(s, buf_a, wsem_a)
        g_start(s + 2, buf_a, gsem_a)

      @pl.when(s + 3 < _SEQ)
      def _():
        w_wait(s + 1, buf_b, wsem_b)
        g_start(s + 3, buf_b, gsem_b)

      return carry

    lax.fori_loop(0, npair, pair_body, 0)

    s_last = _SEQ - 1
    g_wait(s_last, buf_a, gsem_a)
    add_pos(s_last, buf_a)
    w_start(s_last, buf_a, wsem_a)
    w_wait(s_last, buf_a, wsem_a)
    w_wait(s_last - 1, buf_b, wsem_b)

  return k(ids_w, tok_w, pos_pk)


def kernel(input_ids, token_embedding_weight, position_embedding_weight):
  ids = input_ids.astype(jnp.int32)
  ids_w = ids.T.reshape(_SEQ, _NW, _BPW).transpose(1, 0, 2).reshape(-1)
  pos_flat = position_embedding_weight.reshape(-1)
  out_t = _sc_embed(ids_w, token_embedding_weight, pos_flat)
  return out_t.transpose(1, 0, 2)
